# baseline (device time: 44648 ns/iter reference)
import jax
import jax.numpy as jnp
from jax import lax
from jax.experimental import pallas as pl
from jax.experimental.pallas import tpu as pltpu

N_DEV = 4
M_CHUNKS = 2


def kernel(x, w_mat):
    m_per, k = x.shape
    _, n = w_mat.shape
    n_per = n // N_DEV
    m_sub = m_per // M_CHUNKS

    DT_ORDER = [1, 3, 2, 0]
    RECV_DT_ORDER = [1, 3, 2]

    def body(
        x_hbm, w_hbm, out_hbm,
        out_ref, x_ref, w_buf, y_bufs, s_bufs, recv_bufs, recv_s,
        x_sems, w_sems, o_sems, send_sems, recv_sems, ssend_sems, srecv_sems,
    ):
        my = lax.axis_index("i")

        def x_copy(h):
            return pltpu.make_async_copy(
                x_hbm.at[pl.ds(h * m_sub, m_sub), :],
                x_ref.at[pl.ds(h * m_sub, m_sub), :],
                x_sems.at[h],
            )

        for h in range(M_CHUNKS):
            x_copy(h).start()

        flushes = []

        def flush(row):
            idx = len(flushes)
            d = pltpu.make_async_copy(
                out_ref.at[pl.ds(row, m_sub), :],
                out_hbm.at[pl.ds(row, m_sub), :],
                o_sems.at[idx],
            )
            d.start()
            flushes.append(d)

        def w_copy(j, slot):
            tt = (my + DT_ORDER[j]) % N_DEV
            return pltpu.make_async_copy(
                w_hbm.at[:, pl.ds(tt * n_per, n_per)],
                w_buf.at[slot],
                w_sems.at[slot],
            )

        w_copy(0, 0).start()

        bsem = pltpu.get_barrier_semaphore()
        for dt in range(1, N_DEV):
            pl.semaphore_signal(
                bsem, inc=1,
                device_id=((my + dt) % N_DEV,),
                device_id_type=pl.DeviceIdType.MESH,
            )
        barrier_waited = [False]

        def process_recv(dt):
            s = (my - dt) % N_DEV
            for h in range(M_CHUNKS):
                recv = pltpu.make_async_remote_copy(
                    src_ref=y_bufs.at[0, pl.ds(0, m_sub), :],
                    dst_ref=recv_bufs.at[dt - 1, pl.ds(h * m_sub, m_sub), :],
                    send_sem=send_sems.at[dt, h],
                    recv_sem=recv_sems.at[dt, h],
                    device_id=(s,),
                    device_id_type=pl.DeviceIdType.MESH,
                )
                recv.wait_recv()
                srecv = pltpu.make_async_remote_copy(
                    src_ref=s_bufs.at[0, 0, :],
                    dst_ref=recv_s.at[dt - 1, h, :],
                    send_sem=ssend_sems.at[dt, h],
                    recv_sem=srecv_sems.at[dt, h],
                    device_id=(s,),
                    device_id_type=pl.DeviceIdType.MESH,
                )
                srecv.wait_recv()
                q = recv_bufs[dt - 1, pl.ds(h * m_sub, m_sub), :].astype(
                    jnp.float32
                )
                sc = recv_s[dt - 1, h, :]
                out_ref[pl.ds(s * m_per + h * m_sub, m_sub), :] = (
                    q * sc[:, None]
                )
                flush(s * m_per + h * m_sub)

        rdmas = []
        for j in range(N_DEV):
            dt = DT_ORDER[j]
            tt = (my + dt) % N_DEV
            slot = j % 2
            w_copy(j, slot).wait()
            if j + 1 < N_DEV:
                w_copy(j + 1, (j + 1) % 2).start()

            for h in range(M_CHUNKS):
                if j == 0:
                    x_copy(h).wait()
                y = jnp.dot(
                    x_ref[pl.ds(h * m_sub, m_sub), :],
                    w_buf[slot],
                    preferred_element_type=jnp.float32,
                )
                y = y * (1.0 / (1.0 + jnp.exp(-y)))

                if dt == 0:
                    out_ref[pl.ds(my * m_per + h * m_sub, m_sub), :] = y
                    flush(my * m_per + h * m_sub)
                else:
                    amax = jnp.max(jnp.abs(y), axis=1, keepdims=True)
                    inv = 127.0 / jnp.maximum(amax, 1e-30)
                    y_bufs[j, pl.ds(h * m_sub, m_sub), :] = jnp.round(
                        y * inv
                    ).astype(jnp.int8)
                    s_bufs[j, h, :] = (amax * (1.0 / 127.0))[:, 0]
                    if not barrier_waited[0]:
                        pl.semaphore_wait(bsem, N_DEV - 1)
                        barrier_waited[0] = True
                    rdma = pltpu.make_async_remote_copy(
                        src_ref=y_bufs.at[j, pl.ds(h * m_sub, m_sub), :],
                        dst_ref=recv_bufs.at[dt - 1, pl.ds(h * m_sub, m_sub), :],
                        send_sem=send_sems.at[dt, h],
                        recv_sem=recv_sems.at[dt, h],
                        device_id=(tt,),
                        device_id_type=pl.DeviceIdType.MESH,
                    )
                    rdma.start()
                    rdmas.append(rdma)
                    srdma = pltpu.make_async_remote_copy(
                        src_ref=s_bufs.at[j, h, :],
                        dst_ref=recv_s.at[dt - 1, h, :],
                        send_sem=ssend_sems.at[dt, h],
                        recv_sem=srecv_sems.at[dt, h],
                        device_id=(tt,),
                        device_id_type=pl.DeviceIdType.MESH,
                    )
                    srdma.start()
                    rdmas.append(srdma)

            if j == 1:
                process_recv(1)
            elif j == 2:
                process_recv(3)
            elif j == 3:
                process_recv(2)

        for rdma in rdmas:
            rdma.wait_send()
        for d in flushes:
            d.wait()

    return pl.pallas_call(
        body,
        out_shape=jax.ShapeDtypeStruct((N_DEV * m_per, n_per), jnp.float32),
        in_specs=[
            pl.BlockSpec(memory_space=pl.ANY),
            pl.BlockSpec(memory_space=pl.ANY),
        ],
        out_specs=pl.BlockSpec(memory_space=pl.ANY),
        scratch_shapes=[
            pltpu.VMEM((N_DEV * m_per, n_per), jnp.float32),
            pltpu.VMEM((m_per, k), jnp.float32),
            pltpu.VMEM((2, k, n_per), jnp.float32),
            pltpu.VMEM((N_DEV - 1, m_per, n_per), jnp.int8),
            pltpu.VMEM((N_DEV - 1, M_CHUNKS, m_sub), jnp.float32),
            pltpu.VMEM((N_DEV - 1, m_per, n_per), jnp.int8),
            pltpu.VMEM((N_DEV - 1, M_CHUNKS, m_sub), jnp.float32),
            pltpu.SemaphoreType.DMA((M_CHUNKS,)),
            pltpu.SemaphoreType.DMA((2,)),
            pltpu.SemaphoreType.DMA((2 * N_DEV,)),
            pltpu.SemaphoreType.DMA((N_DEV, M_CHUNKS)),
            pltpu.SemaphoreType.DMA((N_DEV, M_CHUNKS)),
            pltpu.SemaphoreType.DMA((N_DEV, M_CHUNKS)),
            pltpu.SemaphoreType.DMA((N_DEV, M_CHUNKS)),
        ],
        compiler_params=pltpu.CompilerParams(
            vmem_limit_bytes=60 * 1024 * 1024,
            collective_id=0,
        ),
    )(x, w_mat)


# device time: 43018 ns/iter; 1.0379x vs baseline; 1.0379x over previous
import jax
import jax.numpy as jnp
from jax import lax
from jax.experimental import pallas as pl
from jax.experimental.pallas import tpu as pltpu

N_DEV = 4
M_CHUNKS = 2


def kernel(x, w_mat):
    m_per, k = x.shape
    _, n = w_mat.shape
    n_per = n // N_DEV
    m_sub = m_per // M_CHUNKS

    DT_ORDER = [1, 3, 2, 0]
    RECV_DT_ORDER = [1, 3, 2]

    def body(
        x_hbm, w_hbm, out_hbm,
        out_ref, x_ref, w_buf, y_bufs, s_bufs, recv_bufs, recv_s,
        x_sems, w_sems, o_sems, send_sems, recv_sems, ssend_sems, srecv_sems,
    ):
        my = lax.axis_index("i")

        def x_copy(h):
            return pltpu.make_async_copy(
                x_hbm.at[pl.ds(h * m_sub, m_sub), :],
                x_ref.at[pl.ds(h * m_sub, m_sub), :],
                x_sems.at[h],
            )

        for h in range(M_CHUNKS):
            x_copy(h).start()

        flushes = []

        def flush(row):
            idx = len(flushes)
            d = pltpu.make_async_copy(
                out_ref.at[pl.ds(row, m_sub), :],
                out_hbm.at[pl.ds(row, m_sub), :],
                o_sems.at[idx],
            )
            d.start()
            flushes.append(d)

        def w_copy(j, slot):
            tt = (my + DT_ORDER[j]) % N_DEV
            return pltpu.make_async_copy(
                w_hbm.at[:, pl.ds(tt * n_per, n_per)],
                w_buf.at[slot],
                w_sems.at[slot],
            )

        w_copy(0, 0).start()

        bsem = pltpu.get_barrier_semaphore()
        for dt in range(1, N_DEV):
            pl.semaphore_signal(
                bsem, inc=1,
                device_id=((my + dt) % N_DEV,),
                device_id_type=pl.DeviceIdType.MESH,
            )
        barrier_waited = [False]

        def process_recv(dt):
            s = (my - dt) % N_DEV
            for h in range(M_CHUNKS):
                recv = pltpu.make_async_remote_copy(
                    src_ref=y_bufs.at[0, pl.ds(0, m_sub), :],
                    dst_ref=recv_bufs.at[dt - 1, pl.ds(h * m_sub, m_sub), :],
                    send_sem=send_sems.at[dt, h],
                    recv_sem=recv_sems.at[dt, h],
                    device_id=(s,),
                    device_id_type=pl.DeviceIdType.MESH,
                )
                recv.wait_recv()
                srecv = pltpu.make_async_remote_copy(
                    src_ref=s_bufs.at[0, 0, :],
                    dst_ref=recv_s.at[dt - 1, h, :],
                    send_sem=ssend_sems.at[dt, h],
                    recv_sem=srecv_sems.at[dt, h],
                    device_id=(s,),
                    device_id_type=pl.DeviceIdType.MESH,
                )
                srecv.wait_recv()
                q = recv_bufs[dt - 1, pl.ds(h * m_sub, m_sub), :].astype(
                    jnp.float32
                )
                sc = recv_s[dt - 1, h, :]
                out_ref[pl.ds(s * m_per + h * m_sub, m_sub), :] = (
                    q * sc[:, None]
                )
                flush(s * m_per + h * m_sub)

        rdmas = []
        for j in range(N_DEV):
            dt = DT_ORDER[j]
            tt = (my + dt) % N_DEV
            slot = j % 2
            w_copy(j, slot).wait()
            if j + 1 < N_DEV:
                w_copy(j + 1, (j + 1) % 2).start()

            for h in range(M_CHUNKS):
                if j == 0:
                    x_copy(h).wait()
                y = jnp.dot(
                    x_ref[pl.ds(h * m_sub, m_sub), :],
                    w_buf[slot],
                    preferred_element_type=jnp.float32,
                )
                y = y * (1.0 / (1.0 + jnp.exp(-y)))

                if dt == 0:
                    out_ref[pl.ds(my * m_per + h * m_sub, m_sub), :] = y
                    flush(my * m_per + h * m_sub)
                else:
                    amax = jnp.max(jnp.abs(y), axis=1, keepdims=True)
                    inv = 127.0 / jnp.maximum(amax, 1e-30)
                    y_bufs[j, pl.ds(h * m_sub, m_sub), :] = jnp.round(
                        y * inv
                    ).astype(jnp.int8)
                    s_bufs[j, h, :] = (amax * (1.0 / 127.0))[:, 0]
                    if not barrier_waited[0]:
                        pl.semaphore_wait(bsem, N_DEV - 1)
                        barrier_waited[0] = True
                    rdma = pltpu.make_async_remote_copy(
                        src_ref=y_bufs.at[j, pl.ds(h * m_sub, m_sub), :],
                        dst_ref=recv_bufs.at[dt - 1, pl.ds(h * m_sub, m_sub), :],
                        send_sem=send_sems.at[dt, h],
                        recv_sem=recv_sems.at[dt, h],
                        device_id=(tt,),
                        device_id_type=pl.DeviceIdType.MESH,
                    )
                    rdma.start()
                    rdmas.append(rdma)
                    srdma = pltpu.make_async_remote_copy(
                        src_ref=s_bufs.at[j, h, :],
                        dst_ref=recv_s.at[dt - 1, h, :],
                        send_sem=ssend_sems.at[dt, h],
                        recv_sem=srecv_sems.at[dt, h],
                        device_id=(tt,),
                        device_id_type=pl.DeviceIdType.MESH,
                    )
                    srdma.start()
                    rdmas.append(srdma)

            if j == 2:
                process_recv(1)
            elif j == 3:
                process_recv(3)

        process_recv(2)

        for rdma in rdmas:
            rdma.wait_send()
        for d in flushes:
            d.wait()

    return pl.pallas_call(
        body,
        out_shape=jax.ShapeDtypeStruct((N_DEV * m_per, n_per), jnp.float32),
        in_specs=[
            pl.BlockSpec(memory_space=pl.ANY),
            pl.BlockSpec(memory_space=pl.ANY),
        ],
        out_specs=pl.BlockSpec(memory_space=pl.ANY),
        scratch_shapes=[
            pltpu.VMEM((N_DEV * m_per, n_per), jnp.float32),
            pltpu.VMEM((m_per, k), jnp.float32),
            pltpu.VMEM((2, k, n_per), jnp.float32),
            pltpu.VMEM((N_DEV - 1, m_per, n_per), jnp.int8),
            pltpu.VMEM((N_DEV - 1, M_CHUNKS, m_sub), jnp.float32),
            pltpu.VMEM((N_DEV - 1, m_per, n_per), jnp.int8),
            pltpu.VMEM((N_DEV - 1, M_CHUNKS, m_sub), jnp.float32),
            pltpu.SemaphoreType.DMA((M_CHUNKS,)),
            pltpu.SemaphoreType.DMA((2,)),
            pltpu.SemaphoreType.DMA((2 * N_DEV,)),
            pltpu.SemaphoreType.DMA((N_DEV, M_CHUNKS)),
            pltpu.SemaphoreType.DMA((N_DEV, M_CHUNKS)),
            pltpu.SemaphoreType.DMA((N_DEV, M_CHUNKS)),
            pltpu.SemaphoreType.DMA((N_DEV, M_CHUNKS)),
        ],
        compiler_params=pltpu.CompilerParams(
            vmem_limit_bytes=60 * 1024 * 1024,
            collective_id=0,
        ),
    )(x, w_mat)
